# Initial kernel scaffold; baseline (speedup 1.0000x reference)
import jax
import jax.numpy as jnp
from jax.experimental import pallas as pl


def kernel(x, w1t, b1, w2t, b2, w3t, b3):
    raise NotImplementedError("write your pallas kernel here")


# speedup vs baseline: 2.5337x; 2.5337x over previous
import jax
import jax.numpy as jnp
from jax.experimental import pallas as pl
from jax.experimental.pallas import tpu as pltpu


def _mlp_kernel(x_ref, w1_ref, b1_ref, w2_ref, b2_ref, w3_ref, b3_ref, o_ref):
    x = x_ref[...].astype(jnp.bfloat16)
    h1 = jnp.dot(x, w1_ref[...], preferred_element_type=jnp.float32)
    h1 = jnp.maximum(h1 + b1_ref[...], 0.0).astype(jnp.bfloat16)
    h2 = jnp.dot(h1, w2_ref[...], preferred_element_type=jnp.float32)
    h2 = jnp.maximum(h2 + b2_ref[...], 0.0).astype(jnp.bfloat16)
    q = jnp.dot(h2, w3_ref[...], preferred_element_type=jnp.float32)
    o_ref[...] = (q + b3_ref[...]).astype(o_ref.dtype)


def kernel(x, w1t, b1, w2t, b2, w3t, b3):
    B = x.shape[0]
    tb = min(4096, B)
    n_blk = pl.cdiv(B, tb)

    w1b = w1t.astype(jnp.bfloat16)
    w2b = w2t.astype(jnp.bfloat16)
    w3b = w3t.astype(jnp.bfloat16)

    const = lambda shape: pl.BlockSpec(shape, lambda i: (0, 0))

    flops = 2 * B * (9 * 128 + 128 * 64 + 64 * 9)
    bytes_accessed = 4 * B * 9 * 2 + 2 * (9 * 128 + 128 * 64 + 64 * 9) \
        + 4 * (128 + 64 + 9)

    return pl.pallas_call(
        _mlp_kernel,
        out_shape=jax.ShapeDtypeStruct((B, 9), x.dtype),
        grid=(n_blk,),
        in_specs=[
            pl.BlockSpec((tb, 9), lambda i: (i, 0)),
            const(w1b.shape), const(b1.shape),
            const(w2b.shape), const(b2.shape),
            const(w3b.shape), const(b3.shape),
        ],
        out_specs=pl.BlockSpec((tb, 9), lambda i: (i, 0)),
        compiler_params=pltpu.CompilerParams(
            dimension_semantics=("parallel",),
        ),
        cost_estimate=pl.CostEstimate(flops=flops, transcendentals=0,
                                      bytes_accessed=bytes_accessed),
    )(x, w1b, b1, w2b, b2, w3b, b3)


# speedup vs baseline: 2.7965x; 1.1037x over previous
import jax
import jax.numpy as jnp
from jax.experimental import pallas as pl
from jax.experimental.pallas import tpu as pltpu


def _mlp_kernel(x_ref, w1_ref, b1_ref, w2_ref, b2_ref, w3_ref, b3_ref, o_ref):
    x = x_ref[...].astype(jnp.bfloat16)
    h1 = jnp.dot(x, w1_ref[...], preferred_element_type=jnp.float32)
    h1 = jnp.maximum(h1 + b1_ref[...], 0.0).astype(jnp.bfloat16)
    h2 = jnp.dot(h1, w2_ref[...], preferred_element_type=jnp.float32)
    h2 = jnp.maximum(h2 + b2_ref[...], 0.0).astype(jnp.bfloat16)
    q = jnp.dot(h2, w3_ref[...], preferred_element_type=jnp.float32)
    o_ref[...] = (q + b3_ref[...]).astype(o_ref.dtype)


def kernel(x, w1t, b1, w2t, b2, w3t, b3):
    B = x.shape[0]
    tb = min(8192, B)
    n_blk = pl.cdiv(B, tb)

    w1b = w1t.astype(jnp.bfloat16)
    w2b = w2t.astype(jnp.bfloat16)
    w3b = w3t.astype(jnp.bfloat16)

    const = lambda shape: pl.BlockSpec(shape, lambda i: (0, 0))

    flops = 2 * B * (9 * 128 + 128 * 64 + 64 * 9)
    bytes_accessed = 4 * B * 9 * 2 + 2 * (9 * 128 + 128 * 64 + 64 * 9) \
        + 4 * (128 + 64 + 9)

    return pl.pallas_call(
        _mlp_kernel,
        out_shape=jax.ShapeDtypeStruct((B, 9), x.dtype),
        grid=(n_blk,),
        in_specs=[
            pl.BlockSpec((tb, 9), lambda i: (i, 0)),
            const(w1b.shape), const(b1.shape),
            const(w2b.shape), const(b2.shape),
            const(w3b.shape), const(b3.shape),
        ],
        out_specs=pl.BlockSpec((tb, 9), lambda i: (i, 0)),
        compiler_params=pltpu.CompilerParams(
            dimension_semantics=("parallel",),
        ),
        cost_estimate=pl.CostEstimate(flops=flops, transcendentals=0,
                                      bytes_accessed=bytes_accessed),
    )(x, w1b, b1, w2b, b2, w3b, b3)
